# Initial kernel scaffold; baseline (speedup 1.0000x reference)
import jax
import jax.numpy as jnp
from jax.experimental import pallas as pl


def kernel(vectors, node_feats, radial_embedding, receivers, W_radial, b_radial):
    raise NotImplementedError("write your pallas kernel here")


# speedup vs baseline: 2.2326x; 2.2326x over previous
import jax
import jax.numpy as jnp
from jax.experimental import pallas as pl
from jax.experimental.pallas import tpu as pltpu

_N = 10000
_DEG = 16
_D = 128
_BS = 40
_E = _BS * _DEG
_NB = _N // _BS

_S3 = 3.0 ** 0.5
_S15 = 15.0 ** 0.5
_S5H = (5.0 ** 0.5) / 2.0
_S15H = _S15 / 2.0


def _mp_kernel(vec_ref, nf_ref, re_ref, recv_ref, wg_ref, bg_ref,
               out_s_ref, out_v_ref, ms_ref, mv_ref):
    nb = pl.program_id(0)

    @pl.when(nb == 0)
    def _init():
        out_s_ref[...] = jnp.zeros_like(out_s_ref)
        out_v_ref[...] = jnp.zeros_like(out_v_ref)

    x = vec_ref[:, 0:1]
    y = vec_ref[:, 1:2]
    z = vec_ref[:, 2:3]
    rn = jnp.sqrt(x * x + y * y + z * z + 1e-9)
    x = x / rn
    y = y / rn
    z = z / rn
    sh = (
        _S3 * x,
        _S3 * y,
        _S3 * z,
        _S15 * x * y,
        _S15 * y * z,
        _S5H * (3.0 * z * z - 1.0),
        _S15 * x * z,
        _S15H * (x * x - y * y),
    )

    re = re_ref[...]
    g = []
    for l in range(3):
        acc = jnp.broadcast_to(bg_ref[pl.ds(l, 1), :], (_E, _D))
        for q in range(8):
            acc = acc + re[:, q:q + 1] * wg_ref[pl.ds(l * 8 + q, 1), :]
        g.append(acc)
    g0, g1, g2 = g

    f = nf_ref[...]
    f_e = jnp.reshape(
        jnp.broadcast_to(f[:, None, :], (_BS, _DEG, _D)), (_E, _D))
    ms_ref[...] = f_e * g0
    fg1 = f_e * g1
    fg2 = f_e * g2
    for j in range(8):
        fg = fg1 if j < 3 else fg2
        mv_ref[:, j, :] = fg * sh[j]

    def body(e, carry):
        r = recv_ref[0, 0, e]
        out_s_ref[pl.ds(r, 1), :] = (
            out_s_ref[pl.ds(r, 1), :] + ms_ref[pl.ds(e, 1), :])
        out_v_ref[pl.ds(r, 1)] = (
            out_v_ref[pl.ds(r, 1)] + mv_ref[pl.ds(e, 1)])
        return carry

    jax.lax.fori_loop(0, _E, body, 0, unroll=2)


def kernel(vectors, node_feats, radial_embedding, receivers, W_radial, b_radial):
    n, d = node_feats.shape
    ve = vectors.reshape(n * _DEG, 3)
    re2 = radial_embedding.reshape(n * _DEG, 8)
    recv3 = receivers.reshape(_NB, 1, _E)
    wg2 = jnp.concatenate(
        [W_radial[:, :d], W_radial[:, d::2], W_radial[:, d + 1::2]], axis=0)
    bg8 = jnp.zeros((8, d), b_radial.dtype)
    bg8 = bg8.at[0].set(b_radial[:d])
    bg8 = bg8.at[1].set(b_radial[d::2])
    bg8 = bg8.at[2].set(b_radial[d + 1::2])

    out_s, out_v = pl.pallas_call(
        _mp_kernel,
        grid=(_NB,),
        in_specs=[
            pl.BlockSpec((_E, 3), lambda i: (i, 0)),
            pl.BlockSpec((_BS, _D), lambda i: (i, 0)),
            pl.BlockSpec((_E, 8), lambda i: (i, 0)),
            pl.BlockSpec((1, 1, _E), lambda i: (i, 0, 0),
                         memory_space=pltpu.SMEM),
            pl.BlockSpec((24, _D), lambda i: (0, 0)),
            pl.BlockSpec((8, _D), lambda i: (0, 0)),
        ],
        out_specs=[
            pl.BlockSpec((_N, _D), lambda i: (0, 0)),
            pl.BlockSpec((_N, 8, _D), lambda i: (0, 0, 0)),
        ],
        out_shape=[
            jax.ShapeDtypeStruct((_N, _D), jnp.float32),
            jax.ShapeDtypeStruct((_N, 8, _D), jnp.float32),
        ],
        scratch_shapes=[
            pltpu.VMEM((_E, _D), jnp.float32),
            pltpu.VMEM((_E, 8, _D), jnp.float32),
        ],
        compiler_params=pltpu.CompilerParams(
            dimension_semantics=("arbitrary",)),
    )(ve, node_feats, re2, recv3, wg2, bg8)

    out_vec = jnp.transpose(out_v, (0, 2, 1)).reshape(n, 8 * d)
    return jnp.concatenate([out_s, out_vec], axis=1)
